# baseline (speedup 1.0000x reference)
import functools

import jax
import jax.numpy as jnp
from jax import lax
from jax.experimental import pallas as pl
from jax.experimental.pallas import tpu as pltpu
from jax.experimental.pallas import tpu_sc as plsc

N_TOTAL = 16 * 2048 * 512
NW = 32
PER_W = N_TOTAL // NW
CHUNK = 16384
NCH = PER_W // CHUNK

_mesh = plsc.VectorSubcoreMesh(core_axis_name="c", subcore_axis_name="s")


@functools.partial(
    pl.kernel,
    mesh=_mesh,
    out_type=jax.ShapeDtypeStruct((NW, 2, 16), jnp.float32),
    scratch_types=[
        pltpu.VMEM((2, CHUNK), jnp.float32),
        pltpu.VMEM((2, CHUNK), jnp.float32),
        pltpu.VMEM((2, 16), jnp.float32),
        pltpu.SemaphoreType.DMA,
        pltpu.SemaphoreType.DMA,
    ],
)
def _masked_sum_sc(msk_hbm, inp_hbm, out_hbm, ibuf, mbuf, obuf, sem0, sem1):
    core = lax.axis_index("c")
    sub = lax.axis_index("s")
    wid = sub * 2 + core
    base = wid * PER_W
    sems = (sem0, sem1)

    def start(g, slot):
        off = base + g * CHUNK
        h1 = pltpu.async_copy(inp_hbm.at[pl.ds(off, CHUNK)], ibuf.at[slot], sems[slot])
        h2 = pltpu.async_copy(msk_hbm.at[pl.ds(off, CHUNK)], mbuf.at[slot], sems[slot])
        return (h1, h2)

    pending = start(0, 0)
    acc_s = jnp.zeros((16,), jnp.float32)
    acc_c = jnp.zeros((16,), jnp.float32)

    for g in range(NCH):
        slot = g % 2
        nxt = start(g + 1, 1 - slot) if g + 1 < NCH else None
        pending[0].wait()
        pending[1].wait()

        def body(i, carry, _slot=slot):
            s, c = carry
            off = pl.multiple_of(i * 64, 64)
            for k in range(4):
                v = ibuf[_slot, pl.ds(off + k * 16, 16)]
                m = mbuf[_slot, pl.ds(off + k * 16, 16)]
                s = s + v * m
                c = c + m
            return (s, c)

        acc_s, acc_c = lax.fori_loop(0, CHUNK // 64, body, (acc_s, acc_c))
        if nxt is not None:
            pending = nxt

    obuf[0, :] = acc_s
    obuf[1, :] = acc_c
    pltpu.sync_copy(obuf, out_hbm.at[wid])


def kernel(mask, input):
    inp = input.reshape(-1)
    mskf = mask.reshape(-1).astype(jnp.float32)
    parts = _masked_sum_sc(mskf, inp)
    total = parts[:, 0, :].sum()
    count = parts[:, 1, :].sum()
    return total / count


# speedup vs baseline: 1.2735x; 1.2735x over previous
import functools

import jax
import jax.numpy as jnp
from jax import lax
from jax.experimental import pallas as pl
from jax.experimental.pallas import tpu as pltpu
from jax.experimental.pallas import tpu_sc as plsc

B, R, C = 16, 2048, 512
S = 1024
RS = R - S
NW = 32
ROWS_W = RS // 2
CHUNK_R = 32
NCH = ROWS_W // CHUNK_R
BR = 256

_mesh = plsc.VectorSubcoreMesh(core_axis_name="c", subcore_axis_name="s")


@functools.partial(
    pl.kernel,
    mesh=_mesh,
    out_type=jax.ShapeDtypeStruct((NW * 32,), jnp.float32),
    scratch_types=[
        pltpu.VMEM((2, CHUNK_R, C), jnp.float32),
        pltpu.VMEM((2, CHUNK_R // 4, C), jnp.int32),
        pltpu.VMEM((32,), jnp.float32),
        pltpu.SemaphoreType.DMA,
        pltpu.SemaphoreType.DMA,
    ],
    compiler_params=pltpu.CompilerParams(use_tc_tiling_on_sc=True),
)
def _masked_sum_sc(mi_hbm, inp_hbm, out_hbm, ibuf, mbuf, obuf, sem0, sem1):
    core = lax.axis_index("c")
    sub = lax.axis_index("s")
    wid = sub * 2 + core
    b = sub
    r_base = S + core * ROWS_W
    m_base = core * (ROWS_W // 4)
    sems = (sem0, sem1)

    def copies(g, slot):
        r0 = r_base + g * CHUNK_R
        m0 = m_base + g * (CHUNK_R // 4)
        return (
            pltpu.make_async_copy(inp_hbm.at[b, pl.ds(r0, CHUNK_R), :], ibuf.at[slot], sems[slot]),
            pltpu.make_async_copy(mi_hbm.at[b, pl.ds(m0, CHUNK_R // 4), :], mbuf.at[slot], sems[slot]),
        )

    def start(g, slot):
        for cp in copies(g, slot):
            cp.start()

    def wait(g, slot):
        for cp in copies(g, slot):
            cp.wait()

    def compute(slot, acc):
        def rowgrp_body(r4, carry):
            s, cnt = carry
            rb = pl.multiple_of(r4 * 4, 4)
            for cb in range(C // 16):
                mw = mbuf[slot, r4, pl.ds(cb * 16, 16)]
                for j in range(4):
                    mf = ((mw >> (8 * j)) & 1).astype(jnp.float32)
                    v = ibuf[slot, rb + j, pl.ds(cb * 16, 16)]
                    s = s + v * mf
                    cnt = cnt + mf
            return (s, cnt)

        return lax.fori_loop(0, CHUNK_R // 4, rowgrp_body, acc)

    start(0, 0)
    start(1, 1)
    acc = (jnp.zeros((16,), jnp.float32), jnp.zeros((16,), jnp.float32))

    def main_body(G, carry):
        for slot in range(2):
            g = 2 * G + slot
            wait(g, slot)
            carry = compute(slot, carry)
            start(g + 2, slot)
        return carry

    acc = lax.fori_loop(0, NCH // 2 - 1, main_body, acc)
    for slot in range(2):
        g = NCH - 2 + slot
        wait(g, slot)
        acc = compute(slot, acc)

    obuf[pl.ds(0, 16)] = acc[0]
    obuf[pl.ds(16, 16)] = acc[1]
    pltpu.sync_copy(obuf, out_hbm.at[pl.ds(wid * 32, 32)])


def _tc_body(m_ref, x_ref, o_ref, acc_ref):
    bi = pl.program_id(0)
    ri = pl.program_id(1)

    @pl.when((bi == 0) & (ri == 0))
    def _():
        acc_ref[0] = 0.0
        acc_ref[1] = 0.0

    m = m_ref[...]
    x = x_ref[...]
    acc_ref[0] += jnp.sum(jnp.where(m, x, 0.0))
    acc_ref[1] += jnp.sum(m.astype(jnp.float32))

    @pl.when((bi == pl.num_programs(0) - 1) & (ri == pl.num_programs(1) - 1))
    def _():
        o_ref[0] = acc_ref[0]
        o_ref[1] = acc_ref[1]


_masked_sum_tc = pl.pallas_call(
    _tc_body,
    grid=(B, S // BR),
    in_specs=[
        pl.BlockSpec((1, BR, C), lambda b, i: (b, i, 0)),
        pl.BlockSpec((1, BR, C), lambda b, i: (b, i, 0)),
    ],
    out_specs=pl.BlockSpec(memory_space=pltpu.SMEM),
    out_shape=jax.ShapeDtypeStruct((2,), jnp.float32),
    scratch_shapes=[pltpu.SMEM((2,), jnp.float32)],
)


def kernel(mask, input):
    mm = mask[:, S:, :].astype(jnp.int32).reshape(B, RS // 4, 4, C)
    weights = jnp.array([1, 1 << 8, 1 << 16, 1 << 24], jnp.int32).reshape(1, 1, 4, 1)
    mi = (mm * weights).sum(axis=2)

    parts = _masked_sum_sc(mi, input).reshape(NW, 2, 16)
    tc = _masked_sum_tc(mask, input)
    total = parts[:, 0, :].sum() + tc[0]
    count = parts[:, 1, :].sum() + tc[1]
    return total / count
